# Initial kernel scaffold; baseline (speedup 1.0000x reference)
import functools

import jax
import jax.numpy as jnp
from jax import lax
from jax.experimental import pallas as pl
from jax.experimental.pallas import tpu as pltpu
from jax.experimental.pallas import tpu_sc as plsc

_N = 10000
_E = 320000
_F = 128
_K = 128
_NC = 2
_NS = 16
_NW = _NC * _NS
_CHUNKS = _E // _K
_BASE_CH = _CHUNKS // _NW
_EXTRA = _CHUNKS - _BASE_CH * _NW
_ROWS = _N // _NS

_mesh = plsc.VectorSubcoreMesh(core_axis_name="c", subcore_axis_name="s")


@functools.partial(
    pl.kernel,
    mesh=_mesh,
    out_type=jax.ShapeDtypeStruct((_NC, _N, 16), jnp.float32),
    scratch_types=[
        pltpu.VMEM((_K,), jnp.int32),
        pltpu.VMEM((_K, 16), jnp.float32),
        pltpu.VMEM_SHARED((_N, 16), jnp.float32),
    ],
)
def _deg(dst_hbm, z16_hbm, ones_hbm, out_hbm, idx_v, ones_v, deg_sh):
    c = lax.axis_index("c")
    s = lax.axis_index("s")
    wid = s * _NC + c
    pltpu.sync_copy(ones_hbm, ones_v)
    pltpu.sync_copy(z16_hbm.at[pl.ds(s * _ROWS, _ROWS)],
                    deg_sh.at[pl.ds(s * _ROWS, _ROWS)])
    plsc.subcore_barrier()
    nch = jnp.where(wid < _EXTRA, _BASE_CH + 1, _BASE_CH)

    def body(i, carry):
        base = (wid + i * _NW) * _K
        pltpu.sync_copy(dst_hbm.at[pl.ds(base, _K)], idx_v)
        pltpu.sync_copy(ones_v, deg_sh.at[idx_v], add=True)
        return carry

    lax.fori_loop(0, nch, body, 0)
    plsc.subcore_barrier()
    pltpu.sync_copy(deg_sh.at[pl.ds(s * _ROWS, _ROWS)],
                    out_hbm.at[c, pl.ds(s * _ROWS, _ROWS)])


@functools.partial(
    pl.kernel,
    mesh=_mesh,
    out_type=jax.ShapeDtypeStruct((_NC, _N, _F), jnp.float32),
    scratch_types=[
        pltpu.VMEM((_K,), jnp.int32),
        pltpu.VMEM((_K,), jnp.int32),
        pltpu.VMEM((_K, _F), jnp.float32),
        pltpu.VMEM_SHARED((_N, _F), jnp.float32),
        pltpu.SemaphoreType.DMA,
    ],
)
def _agg(hs_hbm, src_hbm, dst_hbm, z128_hbm, out_hbm,
         src_v, dst_v, msg_v, agg_sh, sem):
    c = lax.axis_index("c")
    s = lax.axis_index("s")
    wid = s * _NC + c
    pltpu.sync_copy(z128_hbm.at[pl.ds(s * _ROWS, _ROWS)],
                    agg_sh.at[pl.ds(s * _ROWS, _ROWS)])
    plsc.subcore_barrier()
    nch = jnp.where(wid < _EXTRA, _BASE_CH + 1, _BASE_CH)

    def body(i, carry):
        base = (wid + i * _NW) * _K
        pltpu.sync_copy(src_hbm.at[pl.ds(base, _K)], src_v)
        pltpu.async_copy(hs_hbm.at[src_v], msg_v, sem).wait()
        pltpu.sync_copy(dst_hbm.at[pl.ds(base, _K)], dst_v)
        pltpu.sync_copy(msg_v, agg_sh.at[dst_v], add=True)
        return carry

    lax.fori_loop(0, nch, body, 0)
    plsc.subcore_barrier()
    pltpu.sync_copy(agg_sh.at[pl.ds(s * _ROWS, _ROWS)],
                    out_hbm.at[c, pl.ds(s * _ROWS, _ROWS)])


_B = 1000


def _first_body(x_ref, w_ref, d0_ref, d1_ref, hs_ref, hsel_ref, dinv_ref):
    dinv = lax.rsqrt(d0_ref[...] + d1_ref[...] + 1.0)
    h = jnp.dot(x_ref[...], w_ref[...],
                preferred_element_type=jnp.float32,
                precision=lax.Precision.HIGHEST)
    hs_ref[...] = h * dinv
    hsel_ref[...] = h * dinv * dinv
    dinv_ref[...] = dinv


def _mid_body(a0_ref, a1_ref, hsel_ref, dinv_ref, b_ref, w_ref,
              hs_ref, hselo_ref):
    dinv = dinv_ref[...]
    y = jnp.maximum(dinv * (a0_ref[...] + a1_ref[...]) + hsel_ref[...]
                    + b_ref[...], 0.0)
    h = jnp.dot(y, w_ref[...],
                preferred_element_type=jnp.float32,
                precision=lax.Precision.HIGHEST)
    hs_ref[...] = h * dinv
    hselo_ref[...] = h * dinv * dinv


def _final_body(a0_ref, a1_ref, hsel_ref, dinv_ref, b_ref, out_ref):
    z = (dinv_ref[...] * (a0_ref[...] + a1_ref[...]) + hsel_ref[...]
         + b_ref[...])
    m = jnp.max(z, axis=-1, keepdims=True)
    lse = jnp.log(jnp.sum(jnp.exp(z - m), axis=-1, keepdims=True)) + m
    out_ref[...] = z - lse


def _row_spec(width):
    return pl.BlockSpec((_B, width), lambda i: (i, 0))


def _full_spec(shape):
    return pl.BlockSpec(shape, lambda i: (0,) * len(shape))


_first_call = pl.pallas_call(
    _first_body,
    grid=(_N // _B,),
    in_specs=[_row_spec(_F), _full_spec((_F, _F)), _row_spec(1), _row_spec(1)],
    out_specs=[_row_spec(_F), _row_spec(_F), _row_spec(1)],
    out_shape=[
        jax.ShapeDtypeStruct((_N, _F), jnp.float32),
        jax.ShapeDtypeStruct((_N, _F), jnp.float32),
        jax.ShapeDtypeStruct((_N, 1), jnp.float32),
    ],
)

_mid_call = pl.pallas_call(
    _mid_body,
    grid=(_N // _B,),
    in_specs=[_row_spec(_F), _row_spec(_F), _row_spec(_F), _row_spec(1),
              _full_spec((1, _F)), _full_spec((_F, _F))],
    out_specs=[_row_spec(_F), _row_spec(_F)],
    out_shape=[
        jax.ShapeDtypeStruct((_N, _F), jnp.float32),
        jax.ShapeDtypeStruct((_N, _F), jnp.float32),
    ],
)

_final_call = pl.pallas_call(
    _final_body,
    grid=(_N // _B,),
    in_specs=[_row_spec(_F), _row_spec(_F), _row_spec(_F), _row_spec(1),
              _full_spec((1, _F))],
    out_specs=_row_spec(_F),
    out_shape=jax.ShapeDtypeStruct((_N, _F), jnp.float32),
)


def kernel(x, edge_index, W1, b1, W2, b2, W3, b3):
    src = edge_index[0]
    dst = edge_index[1]
    z16 = jnp.zeros((_N, 16), jnp.float32)
    z128 = jnp.zeros((_N, _F), jnp.float32)
    ones16 = jnp.ones((_K, 16), jnp.float32)

    degp = _deg(dst, z16, ones16)
    d0 = degp[0, :, 0:1]
    d1 = degp[1, :, 0:1]

    hs, hsel, dinv = _first_call(x, W1, d0, d1)
    aggp = _agg(hs, src, dst, z128)
    hs, hsel = _mid_call(aggp[0], aggp[1], hsel, dinv, b1.reshape(1, _F), W2)
    aggp = _agg(hs, src, dst, z128)
    hs, hsel = _mid_call(aggp[0], aggp[1], hsel, dinv, b2.reshape(1, _F), W3)
    aggp = _agg(hs, src, dst, z128)
    return _final_call(aggp[0], aggp[1], hsel, dinv, b3.reshape(1, _F))


# speedup vs baseline: 10.0539x; 10.0539x over previous
import functools

import jax
import jax.numpy as jnp
from jax import lax
from jax.experimental import pallas as pl
from jax.experimental.pallas import tpu as pltpu
from jax.experimental.pallas import tpu_sc as plsc

_N = 10000
_E = 320000
_F = 128
_K = 128
_NC = 2
_NS = 16
_NW = _NC * _NS
_CHUNKS = _E // _K
_BASE_CH = _CHUNKS // _NW
_EXTRA = _CHUNKS - _BASE_CH * _NW
_NP = 10240
_ROWS = _NP // _NS


@functools.cache
def _deg_call():
    return pl.kernel(
        _deg_body,
        mesh=plsc.VectorSubcoreMesh(core_axis_name="c", subcore_axis_name="s"),
        out_type=jax.ShapeDtypeStruct((_NC, _NP, _F), jnp.float32),
        scratch_types=[
            pltpu.VMEM((_K,), jnp.int32),
            pltpu.VMEM((_K, _F), jnp.float32),
            pltpu.VMEM_SHARED((_NP, _F), jnp.float32),
        ],
    )


def _deg_body(dst_hbm, z128_hbm, ones_hbm, out_hbm, idx_v, ones_v, deg_sh):
    c = lax.axis_index("c")
    s = lax.axis_index("s")
    wid = s * _NC + c
    pltpu.sync_copy(ones_hbm, ones_v)
    pltpu.sync_copy(z128_hbm.at[pl.ds(s * _ROWS, _ROWS)],
                    deg_sh.at[pl.ds(s * _ROWS, _ROWS)])
    plsc.subcore_barrier()
    nch = jnp.where(wid < _EXTRA, _BASE_CH + 1, _BASE_CH)

    def body(i, carry):
        base = (wid + i * _NW) * _K
        pltpu.sync_copy(dst_hbm.at[pl.ds(base, _K)], idx_v)
        pltpu.sync_copy(ones_v, deg_sh.at[idx_v], add=True)
        return carry

    lax.fori_loop(0, nch, body, 0)
    plsc.subcore_barrier()
    pltpu.sync_copy(deg_sh.at[pl.ds(s * _ROWS, _ROWS)],
                    out_hbm.at[c, pl.ds(s * _ROWS, _ROWS)])


@functools.cache
def _agg_call():
    return pl.kernel(
        _agg_body,
        mesh=plsc.VectorSubcoreMesh(core_axis_name="c", subcore_axis_name="s"),
        out_type=jax.ShapeDtypeStruct((_NC, _NP, _F), jnp.float32),
        scratch_types=[
            pltpu.VMEM((_K,), jnp.int32),
            pltpu.VMEM((_K,), jnp.int32),
            pltpu.VMEM((_K, _F), jnp.float32),
            pltpu.VMEM_SHARED((_NP, _F), jnp.float32),
            pltpu.SemaphoreType.DMA,
        ],
    )


def _agg_body(hs_hbm, src_hbm, dst_hbm, z128_hbm, out_hbm,
              src_v, dst_v, msg_v, agg_sh, sem):
    c = lax.axis_index("c")
    s = lax.axis_index("s")
    wid = s * _NC + c
    pltpu.sync_copy(z128_hbm.at[pl.ds(s * _ROWS, _ROWS)],
                    agg_sh.at[pl.ds(s * _ROWS, _ROWS)])
    plsc.subcore_barrier()
    nch = jnp.where(wid < _EXTRA, _BASE_CH + 1, _BASE_CH)

    def body(i, carry):
        base = (wid + i * _NW) * _K
        pltpu.sync_copy(src_hbm.at[pl.ds(base, _K)], src_v)
        pltpu.async_copy(hs_hbm.at[src_v], msg_v, sem).wait()
        pltpu.sync_copy(dst_hbm.at[pl.ds(base, _K)], dst_v)
        pltpu.sync_copy(msg_v, agg_sh.at[dst_v], add=True)
        return carry

    lax.fori_loop(0, nch, body, 0)
    plsc.subcore_barrier()
    pltpu.sync_copy(agg_sh.at[pl.ds(s * _ROWS, _ROWS)],
                    out_hbm.at[c, pl.ds(s * _ROWS, _ROWS)])


_B = 1000


def _first_body(x_ref, w_ref, d0_ref, d1_ref, hs_ref, hsel_ref, dinv_ref):
    dinv = lax.rsqrt(d0_ref[...] + d1_ref[...] + 1.0)
    h = jnp.dot(x_ref[...], w_ref[...],
                preferred_element_type=jnp.float32,
                precision=lax.Precision.HIGHEST)
    hs_ref[...] = h * dinv
    hsel_ref[...] = h * dinv * dinv
    dinv_ref[...] = dinv


def _mid_body(a0_ref, a1_ref, hsel_ref, dinv_ref, b_ref, w_ref,
              hs_ref, hselo_ref):
    dinv = dinv_ref[...]
    y = jnp.maximum(dinv * (a0_ref[...] + a1_ref[...]) + hsel_ref[...]
                    + b_ref[...], 0.0)
    h = jnp.dot(y, w_ref[...],
                preferred_element_type=jnp.float32,
                precision=lax.Precision.HIGHEST)
    hs_ref[...] = h * dinv
    hselo_ref[...] = h * dinv * dinv


def _final_body(a0_ref, a1_ref, hsel_ref, dinv_ref, b_ref, out_ref):
    z = (dinv_ref[...] * (a0_ref[...] + a1_ref[...]) + hsel_ref[...]
         + b_ref[...])
    m = jnp.max(z, axis=-1, keepdims=True)
    lse = jnp.log(jnp.sum(jnp.exp(z - m), axis=-1, keepdims=True)) + m
    out_ref[...] = z - lse


def _row_spec(width):
    return pl.BlockSpec((_B, width), lambda i: (i, 0))


def _full_spec(shape):
    return pl.BlockSpec(shape, lambda i: (0,) * len(shape))


_first_call = pl.pallas_call(
    _first_body,
    grid=(_N // _B,),
    in_specs=[_row_spec(_F), _full_spec((_F, _F)), _row_spec(1), _row_spec(1)],
    out_specs=[_row_spec(_F), _row_spec(_F), _row_spec(1)],
    out_shape=[
        jax.ShapeDtypeStruct((_N, _F), jnp.float32),
        jax.ShapeDtypeStruct((_N, _F), jnp.float32),
        jax.ShapeDtypeStruct((_N, 1), jnp.float32),
    ],
)

_mid_call = pl.pallas_call(
    _mid_body,
    grid=(_N // _B,),
    in_specs=[_row_spec(_F), _row_spec(_F), _row_spec(_F), _row_spec(1),
              _full_spec((1, _F)), _full_spec((_F, _F))],
    out_specs=[_row_spec(_F), _row_spec(_F)],
    out_shape=[
        jax.ShapeDtypeStruct((_N, _F), jnp.float32),
        jax.ShapeDtypeStruct((_N, _F), jnp.float32),
    ],
)

_final_call = pl.pallas_call(
    _final_body,
    grid=(_N // _B,),
    in_specs=[_row_spec(_F), _row_spec(_F), _row_spec(_F), _row_spec(1),
              _full_spec((1, _F))],
    out_specs=_row_spec(_F),
    out_shape=jax.ShapeDtypeStruct((_N, _F), jnp.float32),
)


def kernel(x, edge_index, W1, b1, W2, b2, W3, b3):
    src = edge_index[0]
    dst = edge_index[1]
    z128 = jnp.zeros((_NP, _F), jnp.float32)
    ones128 = jnp.ones((_K, _F), jnp.float32)

    degp = _deg_call()(dst, z128, ones128)
    d0 = degp[0, :_N, 0:1]
    d1 = degp[1, :_N, 0:1]

    hs, hsel, dinv = _first_call(x, W1, d0, d1)
    aggp = _agg_call()(hs, src, dst, z128)
    hs, hsel = _mid_call(aggp[0, :_N], aggp[1, :_N], hsel, dinv,
                         b1.reshape(1, _F), W2)
    aggp = _agg_call()(hs, src, dst, z128)
    hs, hsel = _mid_call(aggp[0, :_N], aggp[1, :_N], hsel, dinv,
                         b2.reshape(1, _F), W3)
    aggp = _agg_call()(hs, src, dst, z128)
    return _final_call(aggp[0, :_N], aggp[1, :_N], hsel, dinv,
                       b3.reshape(1, _F))


# speedup vs baseline: 14.2952x; 1.4219x over previous
import functools

import jax
import jax.numpy as jnp
from jax import lax
from jax.experimental import pallas as pl
from jax.experimental.pallas import tpu as pltpu
from jax.experimental.pallas import tpu_sc as plsc

_N = 10000
_E = 320000
_F = 128
_K = 128
_NC = 2
_NS = 16
_NW = _NC * _NS
_CHUNKS = _E // _K
_BASE_CH = _CHUNKS // _NW
_EXTRA = _CHUNKS - _BASE_CH * _NW
_NP = 10240
_ROWS = _NP // _NS


@functools.cache
def _deg_call():
    return pl.kernel(
        _deg_body,
        mesh=plsc.VectorSubcoreMesh(core_axis_name="c", subcore_axis_name="s"),
        out_type=jax.ShapeDtypeStruct((_NC, _NP, _F), jnp.float32),
        scratch_types=[
            pltpu.VMEM((_K,), jnp.int32),
            pltpu.VMEM((_K, _F), jnp.float32),
            pltpu.VMEM_SHARED((_NP, _F), jnp.float32),
        ],
    )


def _deg_body(dst_hbm, z128_hbm, ones_hbm, out_hbm, idx_v, ones_v, deg_sh):
    c = lax.axis_index("c")
    s = lax.axis_index("s")
    wid = s * _NC + c
    pltpu.sync_copy(ones_hbm, ones_v)
    pltpu.sync_copy(z128_hbm.at[pl.ds(s * _ROWS, _ROWS)],
                    deg_sh.at[pl.ds(s * _ROWS, _ROWS)])
    plsc.subcore_barrier()
    nch = jnp.where(wid < _EXTRA, _BASE_CH + 1, _BASE_CH)

    def body(i, carry):
        base = (wid + i * _NW) * _K
        pltpu.sync_copy(dst_hbm.at[pl.ds(base, _K)], idx_v)
        pltpu.sync_copy(ones_v, deg_sh.at[idx_v], add=True)
        return carry

    lax.fori_loop(0, nch, body, 0)
    plsc.subcore_barrier()
    pltpu.sync_copy(deg_sh.at[pl.ds(s * _ROWS, _ROWS)],
                    out_hbm.at[c, pl.ds(s * _ROWS, _ROWS)])


@functools.cache
def _agg_call():
    return pl.kernel(
        _agg_body,
        mesh=plsc.VectorSubcoreMesh(core_axis_name="c", subcore_axis_name="s"),
        out_type=jax.ShapeDtypeStruct((_NC, _NP, _F), jnp.float32),
        scratch_types=[
            pltpu.VMEM((2, _K), jnp.int32),
            pltpu.VMEM((2, _K), jnp.int32),
            pltpu.VMEM((2, _K, _F), jnp.float32),
            pltpu.VMEM_SHARED((_NP, _F), jnp.float32),
            pltpu.SemaphoreType.DMA,
            pltpu.SemaphoreType.DMA,
        ],
    )


def _agg_body(hs_hbm, src_hbm, dst_hbm, z128_hbm, out_hbm,
              src_v, dst_v, msg_v, agg_sh, sem0, sem1):
    c = lax.axis_index("c")
    s = lax.axis_index("s")
    wid = s * _NC + c
    sems = (sem0, sem1)
    pltpu.sync_copy(z128_hbm.at[pl.ds(s * _ROWS, _ROWS)],
                    agg_sh.at[pl.ds(s * _ROWS, _ROWS)])
    plsc.subcore_barrier()

    def load_and_gather(i, b):
        base = (wid + i * _NW) * _K
        pltpu.sync_copy(src_hbm.at[pl.ds(base, _K)], src_v.at[b])
        pltpu.sync_copy(dst_hbm.at[pl.ds(base, _K)], dst_v.at[b])
        return pltpu.async_copy(hs_hbm.at[src_v.at[b]], msg_v.at[b], sems[b])

    def drain_and_scatter(b):
        pltpu.make_async_copy(hs_hbm.at[src_v.at[b]], msg_v.at[b],
                              sems[b]).wait()
        pltpu.sync_copy(msg_v.at[b], agg_sh.at[dst_v.at[b]], add=True)

    load_and_gather(0, 0)

    def body(j, carry):
        load_and_gather(2 * j + 1, 1)
        drain_and_scatter(0)

        @pl.when(j < _BASE_CH // 2 - 1)
        def _():
            load_and_gather(2 * j + 2, 0)

        drain_and_scatter(1)
        return carry

    lax.fori_loop(0, _BASE_CH // 2, body, 0)

    @pl.when(wid < _EXTRA)
    def _():
        load_and_gather(_BASE_CH, 0)
        drain_and_scatter(0)

    plsc.subcore_barrier()
    pltpu.sync_copy(agg_sh.at[pl.ds(s * _ROWS, _ROWS)],
                    out_hbm.at[c, pl.ds(s * _ROWS, _ROWS)])


_B = 1000


def _first_body(x_ref, w_ref, d0_ref, d1_ref, hs_ref, hsel_ref, dinv_ref):
    dinv = lax.rsqrt(d0_ref[...] + d1_ref[...] + 1.0)
    h = jnp.dot(x_ref[...], w_ref[...],
                preferred_element_type=jnp.float32,
                precision=lax.Precision.HIGHEST)
    hs_ref[...] = h * dinv
    hsel_ref[...] = h * dinv * dinv
    dinv_ref[...] = dinv


def _mid_body(a0_ref, a1_ref, hsel_ref, dinv_ref, b_ref, w_ref,
              hs_ref, hselo_ref):
    dinv = dinv_ref[...]
    y = jnp.maximum(dinv * (a0_ref[...] + a1_ref[...]) + hsel_ref[...]
                    + b_ref[...], 0.0)
    h = jnp.dot(y, w_ref[...],
                preferred_element_type=jnp.float32,
                precision=lax.Precision.HIGHEST)
    hs_ref[...] = h * dinv
    hselo_ref[...] = h * dinv * dinv


def _final_body(a0_ref, a1_ref, hsel_ref, dinv_ref, b_ref, out_ref):
    z = (dinv_ref[...] * (a0_ref[...] + a1_ref[...]) + hsel_ref[...]
         + b_ref[...])
    m = jnp.max(z, axis=-1, keepdims=True)
    lse = jnp.log(jnp.sum(jnp.exp(z - m), axis=-1, keepdims=True)) + m
    out_ref[...] = z - lse


def _row_spec(width):
    return pl.BlockSpec((_B, width), lambda i: (i, 0))


def _full_spec(shape):
    return pl.BlockSpec(shape, lambda i: (0,) * len(shape))


_first_call = pl.pallas_call(
    _first_body,
    grid=(_N // _B,),
    in_specs=[_row_spec(_F), _full_spec((_F, _F)), _row_spec(1), _row_spec(1)],
    out_specs=[_row_spec(_F), _row_spec(_F), _row_spec(1)],
    out_shape=[
        jax.ShapeDtypeStruct((_N, _F), jnp.float32),
        jax.ShapeDtypeStruct((_N, _F), jnp.float32),
        jax.ShapeDtypeStruct((_N, 1), jnp.float32),
    ],
)

_mid_call = pl.pallas_call(
    _mid_body,
    grid=(_N // _B,),
    in_specs=[_row_spec(_F), _row_spec(_F), _row_spec(_F), _row_spec(1),
              _full_spec((1, _F)), _full_spec((_F, _F))],
    out_specs=[_row_spec(_F), _row_spec(_F)],
    out_shape=[
        jax.ShapeDtypeStruct((_N, _F), jnp.float32),
        jax.ShapeDtypeStruct((_N, _F), jnp.float32),
    ],
)

_final_call = pl.pallas_call(
    _final_body,
    grid=(_N // _B,),
    in_specs=[_row_spec(_F), _row_spec(_F), _row_spec(_F), _row_spec(1),
              _full_spec((1, _F))],
    out_specs=_row_spec(_F),
    out_shape=jax.ShapeDtypeStruct((_N, _F), jnp.float32),
)


def kernel(x, edge_index, W1, b1, W2, b2, W3, b3):
    src = edge_index[0]
    dst = edge_index[1]
    z128 = jnp.zeros((_NP, _F), jnp.float32)
    ones128 = jnp.ones((_K, _F), jnp.float32)

    degp = _deg_call()(dst, z128, ones128)
    d0 = degp[0, :_N, 0:1]
    d1 = degp[1, :_N, 0:1]

    hs, hsel, dinv = _first_call(x, W1, d0, d1)
    aggp = _agg_call()(hs, src, dst, z128)
    hs, hsel = _mid_call(aggp[0, :_N], aggp[1, :_N], hsel, dinv,
                         b1.reshape(1, _F), W2)
    aggp = _agg_call()(hs, src, dst, z128)
    hs, hsel = _mid_call(aggp[0, :_N], aggp[1, :_N], hsel, dinv,
                         b2.reshape(1, _F), W3)
    aggp = _agg_call()(hs, src, dst, z128)
    return _final_call(aggp[0, :_N], aggp[1, :_N], hsel, dinv,
                       b3.reshape(1, _F))


# speedup vs baseline: 16.5708x; 1.1592x over previous
import functools

import jax
import jax.numpy as jnp
from jax import lax
from jax.experimental import pallas as pl
from jax.experimental.pallas import tpu as pltpu
from jax.experimental.pallas import tpu_sc as plsc

_N = 10000
_E = 320000
_F = 128
_K = 128
_NC = 2
_NS = 16
_NW = _NC * _NS
_DCPB = 4
_DNBLK = _E // (_K * _DCPB)
_DBPT = _DNBLK // _NW
_DXTRA = _DNBLK - _DBPT * _NW
_ACPB = 2
_ANBLK = _E // (_K * _ACPB)
_ABPT = _ANBLK // _NW
_AXTRA = _ANBLK - _ABPT * _NW
_NP = 10240
_ROWS = _NP // _NS


@functools.cache
def _deg_call():
    return pl.kernel(
        _deg_body,
        mesh=plsc.VectorSubcoreMesh(core_axis_name="c", subcore_axis_name="s"),
        out_type=jax.ShapeDtypeStruct((_NC, _NP), jnp.float32),
        scratch_types=[
            pltpu.VMEM((_DCPB, _K), jnp.int32),
            pltpu.VMEM((_DCPB, _K), jnp.int32),
            pltpu.VMEM((_K,), jnp.float32),
            pltpu.VMEM_SHARED((_NP,), jnp.float32),
            pltpu.SemaphoreType.DMA,
            pltpu.SemaphoreType.DMA,
            pltpu.SemaphoreType.DMA,
            pltpu.SemaphoreType.DMA,
        ],
    )


def _deg_body(dst3_hbm, z1_hbm, ones1_hbm, out_hbm,
              dA, dB, ones_v, deg_sh, s0, s1, s2, s3):
    c = lax.axis_index("c")
    s = lax.axis_index("s")
    wid = s * _NC + c
    ssems = (s0, s1, s2, s3)
    pltpu.sync_copy(ones1_hbm, ones_v)
    pltpu.sync_copy(z1_hbm.at[pl.ds(s * _ROWS, _ROWS)],
                    deg_sh.at[pl.ds(s * _ROWS, _ROWS)])
    plsc.subcore_barrier()
    npairs = jnp.where(wid < _DXTRA, (_DBPT + 1) // 2, (_DBPT + 1) // 2 - 1)

    def blkid(i):
        return wid + i * _NW

    def fire(ebuf, j):
        pltpu.async_copy(ones_v, deg_sh.at[ebuf.at[j]], ssems[j], add=True)

    def drain(ebuf, j):
        pltpu.make_async_copy(ones_v, deg_sh.at[ebuf.at[j]], ssems[j]).wait()

    pltpu.sync_copy(dst3_hbm.at[blkid(0)], dA)

    def pair_body(p, carry):
        @pl.when(p > 0)
        def _():
            for j in range(_DCPB):
                drain(dB, j)
        for j in range(_DCPB):
            fire(dA, j)
        pltpu.sync_copy(dst3_hbm.at[blkid(2 * p + 1)], dB)
        for j in range(_DCPB):
            drain(dA, j)
            fire(dB, j)

        @pl.when(p + 1 < npairs)
        def _():
            pltpu.sync_copy(dst3_hbm.at[blkid(2 * p + 2)], dA)
        return carry

    lax.fori_loop(0, npairs, pair_body, 0)
    for j in range(_DCPB):
        drain(dB, j)

    @pl.when(wid >= _DXTRA)
    def _():
        pltpu.sync_copy(dst3_hbm.at[blkid(_DBPT - 1)], dA)
        for j in range(_DCPB):
            fire(dA, j)
        for j in range(_DCPB):
            drain(dA, j)

    plsc.subcore_barrier()
    pltpu.sync_copy(deg_sh.at[pl.ds(s * _ROWS, _ROWS)],
                    out_hbm.at[c, pl.ds(s * _ROWS, _ROWS)])


@functools.cache
def _agg_call():
    return pl.kernel(
        _agg_body,
        mesh=plsc.VectorSubcoreMesh(core_axis_name="c", subcore_axis_name="s"),
        out_type=jax.ShapeDtypeStruct((_NC, _NP, _F), jnp.float32),
        scratch_types=[
            pltpu.VMEM((2 * _ACPB, _K), jnp.int32),
            pltpu.VMEM((2 * _ACPB, _K), jnp.int32),
            pltpu.VMEM((_ACPB, _K, _F), jnp.float32),
            pltpu.VMEM_SHARED((_NP, _F), jnp.float32),
            pltpu.SemaphoreType.DMA,
            pltpu.SemaphoreType.DMA,
            pltpu.SemaphoreType.DMA,
            pltpu.SemaphoreType.DMA,
        ],
    )


def _agg_body(hs_hbm, eidx_hbm, z128_hbm, out_hbm,
              eA, eB, msg_v, agg_sh, g0, g1, s0, s1):
    c = lax.axis_index("c")
    s = lax.axis_index("s")
    wid = s * _NC + c
    gsems = (g0, g1)
    ssems = (s0, s1)
    pltpu.sync_copy(z128_hbm.at[pl.ds(s * _ROWS, _ROWS)],
                    agg_sh.at[pl.ds(s * _ROWS, _ROWS)])
    plsc.subcore_barrier()
    npairs = jnp.where(wid < _AXTRA, (_ABPT + 1) // 2, (_ABPT + 1) // 2 - 1)

    def blkid(i):
        return wid + i * _NW

    def gfire(ebuf, j):
        pltpu.async_copy(hs_hbm.at[ebuf.at[2 * j]], msg_v.at[j], gsems[j])

    def gdrain(ebuf, j):
        pltpu.make_async_copy(hs_hbm.at[ebuf.at[2 * j]], msg_v.at[j],
                              gsems[j]).wait()

    def sfire(ebuf, j):
        pltpu.async_copy(msg_v.at[j], agg_sh.at[ebuf.at[2 * j + 1]],
                         ssems[j], add=True)

    def sdrain(ebuf, j):
        pltpu.make_async_copy(msg_v.at[j], agg_sh.at[ebuf.at[2 * j + 1]],
                              ssems[j]).wait()

    pltpu.sync_copy(eidx_hbm.at[blkid(0)], eA)

    def pair_body(p, carry):
        @pl.when(p > 0)
        def _():
            for j in range(_ACPB):
                sdrain(eB, j)
        for j in range(_ACPB):
            gfire(eA, j)
        pltpu.sync_copy(eidx_hbm.at[blkid(2 * p + 1)], eB)
        for j in range(_ACPB):
            gdrain(eA, j)
            sfire(eA, j)
        for j in range(_ACPB):
            sdrain(eA, j)
            gfire(eB, j)

        @pl.when(p + 1 < npairs)
        def _():
            pltpu.sync_copy(eidx_hbm.at[blkid(2 * p + 2)], eA)
        for j in range(_ACPB):
            gdrain(eB, j)
            sfire(eB, j)
        return carry

    lax.fori_loop(0, npairs, pair_body, 0)
    for j in range(_ACPB):
        sdrain(eB, j)

    @pl.when(wid >= _AXTRA)
    def _():
        pltpu.sync_copy(eidx_hbm.at[blkid(_ABPT - 1)], eA)
        for j in range(_ACPB):
            gfire(eA, j)
        for j in range(_ACPB):
            gdrain(eA, j)
            sfire(eA, j)
        for j in range(_ACPB):
            sdrain(eA, j)

    plsc.subcore_barrier()
    pltpu.sync_copy(agg_sh.at[pl.ds(s * _ROWS, _ROWS)],
                    out_hbm.at[c, pl.ds(s * _ROWS, _ROWS)])


_B = 1000


def _mm_body(x_ref, w_ref, h_ref):
    h_ref[...] = jnp.dot(x_ref[...], w_ref[...],
                         preferred_element_type=jnp.float32,
                         precision=lax.Precision.HIGHEST)


def _scale_body(h_ref, d0_ref, d1_ref, hs_ref, hsel_ref, dinv_ref):
    dinv = lax.rsqrt(d0_ref[...] + d1_ref[...] + 1.0)
    h = h_ref[...]
    hs_ref[...] = h * dinv
    hsel_ref[...] = h * dinv * dinv
    dinv_ref[...] = dinv


def _mid_body(a0_ref, a1_ref, hsel_ref, dinv_ref, b_ref, w_ref,
              hs_ref, hselo_ref):
    dinv = dinv_ref[...]
    y = jnp.maximum(dinv * (a0_ref[...] + a1_ref[...]) + hsel_ref[...]
                    + b_ref[...], 0.0)
    h = jnp.dot(y, w_ref[...],
                preferred_element_type=jnp.float32,
                precision=lax.Precision.HIGHEST)
    hs_ref[...] = h * dinv
    hselo_ref[...] = h * dinv * dinv


def _final_body(a0_ref, a1_ref, hsel_ref, dinv_ref, b_ref, out_ref):
    z = (dinv_ref[...] * (a0_ref[...] + a1_ref[...]) + hsel_ref[...]
         + b_ref[...])
    m = jnp.max(z, axis=-1, keepdims=True)
    lse = jnp.log(jnp.sum(jnp.exp(z - m), axis=-1, keepdims=True)) + m
    out_ref[...] = z - lse


def _row_spec(width):
    return pl.BlockSpec((_B, width), lambda i: (i, 0))


def _full_spec(shape):
    return pl.BlockSpec(shape, lambda i: (0,) * len(shape))


_mm_call = pl.pallas_call(
    _mm_body,
    grid=(_N // _B,),
    in_specs=[_row_spec(_F), _full_spec((_F, _F))],
    out_specs=_row_spec(_F),
    out_shape=jax.ShapeDtypeStruct((_N, _F), jnp.float32),
)

_scale_call = pl.pallas_call(
    _scale_body,
    grid=(_N // _B,),
    in_specs=[_row_spec(_F), _row_spec(1), _row_spec(1)],
    out_specs=[_row_spec(_F), _row_spec(_F), _row_spec(1)],
    out_shape=[
        jax.ShapeDtypeStruct((_N, _F), jnp.float32),
        jax.ShapeDtypeStruct((_N, _F), jnp.float32),
        jax.ShapeDtypeStruct((_N, 1), jnp.float32),
    ],
)

_mid_call = pl.pallas_call(
    _mid_body,
    grid=(_N // _B,),
    in_specs=[_row_spec(_F), _row_spec(_F), _row_spec(_F), _row_spec(1),
              _full_spec((1, _F)), _full_spec((_F, _F))],
    out_specs=[_row_spec(_F), _row_spec(_F)],
    out_shape=[
        jax.ShapeDtypeStruct((_N, _F), jnp.float32),
        jax.ShapeDtypeStruct((_N, _F), jnp.float32),
    ],
)

_final_call = pl.pallas_call(
    _final_body,
    grid=(_N // _B,),
    in_specs=[_row_spec(_F), _row_spec(_F), _row_spec(_F), _row_spec(1),
              _full_spec((1, _F))],
    out_specs=_row_spec(_F),
    out_shape=jax.ShapeDtypeStruct((_N, _F), jnp.float32),
)


def kernel(x, edge_index, W1, b1, W2, b2, W3, b3):
    src = edge_index[0]
    dst = edge_index[1]
    srcm = src.reshape(_E // _K, _K)
    dstm = dst.reshape(_E // _K, _K)
    eidx3 = jnp.stack([srcm, dstm], axis=1).reshape(_ANBLK, 2 * _ACPB, _K)
    dst3 = dstm.reshape(_DNBLK, _DCPB, _K)
    z1 = jnp.zeros((_NP,), jnp.float32)
    z128 = jnp.zeros((_NP, _F), jnp.float32)
    ones1 = jnp.ones((_K,), jnp.float32)

    degp = _deg_call()(dst3, z1, ones1)
    h1 = _mm_call(x, W1)
    d0 = degp[0, :_N, None]
    d1 = degp[1, :_N, None]
    hs, hsel, dinv = _scale_call(h1, d0, d1)

    aggp = _agg_call()(hs, eidx3, z128)
    hs, hsel = _mid_call(aggp[0, :_N], aggp[1, :_N], hsel, dinv,
                         b1.reshape(1, _F), W2)
    aggp = _agg_call()(hs, eidx3, z128)
    hs, hsel = _mid_call(aggp[0, :_N], aggp[1, :_N], hsel, dinv,
                         b2.reshape(1, _F), W3)
    aggp = _agg_call()(hs, eidx3, z128)
    return _final_call(aggp[0, :_N], aggp[1, :_N], hsel, dinv,
                       b3.reshape(1, _F))


# speedup vs baseline: 17.2109x; 1.0386x over previous
import functools

import jax
import jax.numpy as jnp
from jax import lax
from jax.experimental import pallas as pl
from jax.experimental.pallas import tpu as pltpu
from jax.experimental.pallas import tpu_sc as plsc

_N = 10000
_E = 320000
_F = 128
_K = 128
_NC = 2
_NS = 16
_NW = _NC * _NS
_DCPB = 4
_DNBLK = _E // (_K * _DCPB)
_DBPT = _DNBLK // _NW
_DXTRA = _DNBLK - _DBPT * _NW
_ACPB = 2
_ANBLK = _E // (_K * _ACPB)
_ABPT = _ANBLK // _NW
_AXTRA = _ANBLK - _ABPT * _NW
_NP = 10240
_ROWS = _NP // _NS


@functools.cache
def _deg_call():
    return pl.kernel(
        _deg_body,
        mesh=plsc.VectorSubcoreMesh(core_axis_name="c", subcore_axis_name="s"),
        out_type=jax.ShapeDtypeStruct((_NC, _NP), jnp.float32),
        scratch_types=[
            pltpu.VMEM((_DCPB, _K), jnp.int32),
            pltpu.VMEM((_DCPB, _K), jnp.int32),
            pltpu.VMEM((_K,), jnp.float32),
            pltpu.VMEM_SHARED((_NP,), jnp.float32),
            pltpu.SemaphoreType.DMA,
            pltpu.SemaphoreType.DMA,
            pltpu.SemaphoreType.DMA,
            pltpu.SemaphoreType.DMA,
        ],
    )


def _deg_body(dst3_hbm, z1_hbm, ones1_hbm, out_hbm,
              dA, dB, ones_v, deg_sh, s0, s1, s2, s3):
    c = lax.axis_index("c")
    s = lax.axis_index("s")
    wid = s * _NC + c
    ssems = (s0, s1, s2, s3)
    pltpu.sync_copy(ones1_hbm, ones_v)
    pltpu.sync_copy(z1_hbm.at[pl.ds(s * _ROWS, _ROWS)],
                    deg_sh.at[pl.ds(s * _ROWS, _ROWS)])
    plsc.subcore_barrier()
    npairs = jnp.where(wid < _DXTRA, (_DBPT + 1) // 2, (_DBPT + 1) // 2 - 1)

    def blkid(i):
        return wid + i * _NW

    def fire(ebuf, j):
        pltpu.async_copy(ones_v, deg_sh.at[ebuf.at[j]], ssems[j], add=True)

    def drain(ebuf, j):
        pltpu.make_async_copy(ones_v, deg_sh.at[ebuf.at[j]], ssems[j]).wait()

    pltpu.sync_copy(dst3_hbm.at[blkid(0)], dA)

    def pair_body(p, carry):
        @pl.when(p > 0)
        def _():
            for j in range(_DCPB):
                drain(dB, j)
        for j in range(_DCPB):
            fire(dA, j)
        pltpu.sync_copy(dst3_hbm.at[blkid(2 * p + 1)], dB)
        for j in range(_DCPB):
            drain(dA, j)
            fire(dB, j)

        @pl.when(p + 1 < npairs)
        def _():
            pltpu.sync_copy(dst3_hbm.at[blkid(2 * p + 2)], dA)
        return carry

    lax.fori_loop(0, npairs, pair_body, 0)
    for j in range(_DCPB):
        drain(dB, j)

    @pl.when(wid >= _DXTRA)
    def _():
        pltpu.sync_copy(dst3_hbm.at[blkid(_DBPT - 1)], dA)
        for j in range(_DCPB):
            fire(dA, j)
        for j in range(_DCPB):
            drain(dA, j)

    plsc.subcore_barrier()
    pltpu.sync_copy(deg_sh.at[pl.ds(s * _ROWS, _ROWS)],
                    out_hbm.at[c, pl.ds(s * _ROWS, _ROWS)])


@functools.cache
def _agg_call():
    return pl.kernel(
        _agg_body,
        mesh=plsc.VectorSubcoreMesh(core_axis_name="c", subcore_axis_name="s"),
        out_type=jax.ShapeDtypeStruct((_NC, _NP, _F), jnp.float32),
        scratch_types=[
            pltpu.VMEM((2 * _ACPB, _K), jnp.int32),
            pltpu.VMEM((2 * _ACPB, _K), jnp.int32),
            pltpu.VMEM((_ACPB, _K, _F), jnp.float32),
            pltpu.VMEM_SHARED((_NP, _F), jnp.float32),
            pltpu.SemaphoreType.DMA,
            pltpu.SemaphoreType.DMA,
            pltpu.SemaphoreType.DMA,
            pltpu.SemaphoreType.DMA,
        ],
    )


def _agg_body(hs_hbm, eidx_hbm, z128_hbm, out_hbm,
              eA, eB, msg_v, agg_sh, g0, g1, s0, s1):
    c = lax.axis_index("c")
    s = lax.axis_index("s")
    wid = s * _NC + c
    gsems = (g0, g1)
    ssems = (s0, s1)
    pltpu.sync_copy(z128_hbm.at[pl.ds(s * _ROWS, _ROWS)],
                    agg_sh.at[pl.ds(s * _ROWS, _ROWS)])
    plsc.subcore_barrier()
    npairs = jnp.where(wid < _AXTRA, (_ABPT + 1) // 2, (_ABPT + 1) // 2 - 1)

    def blkid(i):
        return wid + i * _NW

    def gfire(ebuf, j):
        pltpu.async_copy(hs_hbm.at[ebuf.at[2 * j]], msg_v.at[j], gsems[j])

    def gdrain(ebuf, j):
        pltpu.make_async_copy(hs_hbm.at[ebuf.at[2 * j]], msg_v.at[j],
                              gsems[j]).wait()

    def sfire(ebuf, j):
        pltpu.async_copy(msg_v.at[j], agg_sh.at[ebuf.at[2 * j + 1]],
                         ssems[j], add=True)

    def sdrain(ebuf, j):
        pltpu.make_async_copy(msg_v.at[j], agg_sh.at[ebuf.at[2 * j + 1]],
                              ssems[j]).wait()

    pltpu.sync_copy(eidx_hbm.at[blkid(0)], eA)

    def pair_body(p, carry):
        @pl.when(p > 0)
        def _():
            for j in range(_ACPB):
                sdrain(eB, j)
        for j in range(_ACPB):
            gfire(eA, j)
        pltpu.sync_copy(eidx_hbm.at[blkid(2 * p + 1)], eB)
        for j in range(_ACPB):
            gdrain(eA, j)
            sfire(eA, j)
        for j in range(_ACPB):
            sdrain(eA, j)
            gfire(eB, j)

        @pl.when(p + 1 < npairs)
        def _():
            pltpu.sync_copy(eidx_hbm.at[blkid(2 * p + 2)], eA)
        for j in range(_ACPB):
            gdrain(eB, j)
            sfire(eB, j)
        return carry

    lax.fori_loop(0, npairs, pair_body, 0)
    for j in range(_ACPB):
        sdrain(eB, j)

    @pl.when(wid >= _AXTRA)
    def _():
        pltpu.sync_copy(eidx_hbm.at[blkid(_ABPT - 1)], eA)
        for j in range(_ACPB):
            gfire(eA, j)
        for j in range(_ACPB):
            gdrain(eA, j)
            sfire(eA, j)
        for j in range(_ACPB):
            sdrain(eA, j)

    plsc.subcore_barrier()
    pltpu.sync_copy(agg_sh.at[pl.ds(s * _ROWS, _ROWS)],
                    out_hbm.at[c, pl.ds(s * _ROWS, _ROWS)])


_B = 1000


def _mm_body(x_ref, w_ref, h_ref):
    h_ref[...] = jnp.dot(x_ref[...], w_ref[...],
                         preferred_element_type=jnp.float32,
                         precision=lax.Precision.HIGHEST)


def _scale_body(h_ref, d0_ref, d1_ref, hs_ref, hsel_ref, dinv_ref):
    dinv = lax.rsqrt(d0_ref[0] + d1_ref[0] + 1.0)
    h = h_ref[...]
    hs_ref[...] = h * dinv
    hsel_ref[...] = h * dinv * dinv
    dinv_ref[...] = dinv


def _mid_body(a0_ref, a1_ref, hsel_ref, dinv_ref, b_ref, w_ref,
              hs_ref, hselo_ref):
    dinv = dinv_ref[...]
    y = jnp.maximum(dinv * (a0_ref[0] + a1_ref[0]) + hsel_ref[...]
                    + b_ref[...], 0.0)
    h = jnp.dot(y, w_ref[...],
                preferred_element_type=jnp.float32,
                precision=lax.Precision.HIGHEST)
    hs_ref[...] = h * dinv
    hselo_ref[...] = h * dinv * dinv


def _final_body(a0_ref, a1_ref, hsel_ref, dinv_ref, b_ref, out_ref):
    z = (dinv_ref[...] * (a0_ref[0] + a1_ref[0]) + hsel_ref[...]
         + b_ref[...])
    m = jnp.max(z, axis=-1, keepdims=True)
    lse = jnp.log(jnp.sum(jnp.exp(z - m), axis=-1, keepdims=True)) + m
    out_ref[...] = z - lse


def _row_spec(width):
    return pl.BlockSpec((_B, width), lambda i: (i, 0))


def _part_spec(width, part):
    return pl.BlockSpec((1, _B, width), lambda i, _p=part: (_p, i, 0))


def _full_spec(shape):
    return pl.BlockSpec(shape, lambda i: (0,) * len(shape))


_mm_call = pl.pallas_call(
    _mm_body,
    grid=(_N // _B,),
    in_specs=[_row_spec(_F), _full_spec((_F, _F))],
    out_specs=_row_spec(_F),
    out_shape=jax.ShapeDtypeStruct((_N, _F), jnp.float32),
)

_scale_call = pl.pallas_call(
    _scale_body,
    grid=(_N // _B,),
    in_specs=[_row_spec(_F), _part_spec(1, 0), _part_spec(1, 1)],
    out_specs=[_row_spec(_F), _row_spec(_F), _row_spec(1)],
    out_shape=[
        jax.ShapeDtypeStruct((_N, _F), jnp.float32),
        jax.ShapeDtypeStruct((_N, _F), jnp.float32),
        jax.ShapeDtypeStruct((_N, 1), jnp.float32),
    ],
)

_mid_call = pl.pallas_call(
    _mid_body,
    grid=(_N // _B,),
    in_specs=[_part_spec(_F, 0), _part_spec(_F, 1), _row_spec(_F),
              _row_spec(1), _full_spec((1, _F)), _full_spec((_F, _F))],
    out_specs=[_row_spec(_F), _row_spec(_F)],
    out_shape=[
        jax.ShapeDtypeStruct((_N, _F), jnp.float32),
        jax.ShapeDtypeStruct((_N, _F), jnp.float32),
    ],
)

_final_call = pl.pallas_call(
    _final_body,
    grid=(_N // _B,),
    in_specs=[_part_spec(_F, 0), _part_spec(_F, 1), _row_spec(_F),
              _row_spec(1), _full_spec((1, _F))],
    out_specs=_row_spec(_F),
    out_shape=jax.ShapeDtypeStruct((_N, _F), jnp.float32),
)


def kernel(x, edge_index, W1, b1, W2, b2, W3, b3):
    src = edge_index[0]
    dst = edge_index[1]
    srcm = src.reshape(_E // _K, _K)
    dstm = dst.reshape(_E // _K, _K)
    eidx3 = jnp.stack([srcm, dstm], axis=1).reshape(_ANBLK, 2 * _ACPB, _K)
    dst3 = dstm.reshape(_DNBLK, _DCPB, _K)
    z1 = jnp.zeros((_NP,), jnp.float32)
    z128 = jnp.zeros((_NP, _F), jnp.float32)
    ones1 = jnp.ones((_K,), jnp.float32)

    degp = _deg_call()(dst3, z1, ones1)
    h1 = _mm_call(x, W1)
    degp3 = degp[:, :, None]
    hs, hsel, dinv = _scale_call(h1, degp3, degp3)

    aggp = _agg_call()(hs, eidx3, z128)
    hs, hsel = _mid_call(aggp, aggp, hsel, dinv, b1.reshape(1, _F), W2)
    aggp = _agg_call()(hs, eidx3, z128)
    hs, hsel = _mid_call(aggp, aggp, hsel, dinv, b2.reshape(1, _F), W3)
    aggp = _agg_call()(hs, eidx3, z128)
    return _final_call(aggp, aggp, hsel, dinv, b3.reshape(1, _F))
